# baseline (speedup 1.0000x reference)
import functools
import math

import numpy as np
import jax
import jax.numpy as jnp
from jax import lax
from jax.experimental import pallas as pl
from jax.experimental.pallas import tpu as pltpu
from jax.experimental.pallas import tpu_sc as plsc

PS = 16
MASK_RATIO = 0.75
MIN_MASK = 4
MAX_MASK = 48

B, C, H, W = 4, 96, 384, 384
NPH, NPW = H // PS, W // PS
TOTAL = NPH * NPW

_num_masked = max(MIN_MASK, min(int(TOTAL * MASK_RATIO), MAX_MASK))
_bs = int(math.sqrt(_num_masked))
_ch, _cw = NPH // 2, NPW // 2
_MASK_IDS = [i * NPW + j
             for i in range(max(0, _ch - _bs // 2), min(NPH, _ch + _bs // 2))
             for j in range(max(0, _cw - _bs // 2), min(NPW, _cw + _bs // 2))]
_mask_row = np.zeros(TOTAL, dtype=bool)
_mask_row[_MASK_IDS] = True
_KEEP = np.nonzero(~_mask_row)[0]
NKEEP = len(_KEEP)

_mi = np.asarray(_MASK_IDS) // NPW
_mj = np.asarray(_MASK_IDS) % NPW
R0, R1 = int(_mi.min()) * PS, (int(_mi.max()) + 1) * PS
C0, C1 = int(_mj.min()) * PS, (int(_mj.max()) + 1) * PS

_MASK_CONST = np.tile(_mask_row[None, :], (B, 1))

NROWS = B * NKEEP * C * PS
NW = 32
RPW = NROWS // NW
IDX_ROWS_PER_STEP = 15
ROWS_PER_STEP = IDX_ROWS_PER_STEP * 128
STEPS = RPW // ROWS_PER_STEP
assert RPW % ROWS_PER_STEP == 0


def _gather_index() -> np.ndarray:
    ki = _KEEP // NPW
    kj = _KEEP % NPW
    b = np.arange(B)[:, None, None, None]
    i = ki[None, :, None, None]
    j = kj[None, :, None, None]
    c = np.arange(C)[None, None, :, None]
    pi = np.arange(PS)[None, None, None, :]
    src = ((b * C + c) * H + i * PS + pi) * NPW + j
    return src.astype(np.int32).reshape(NW, STEPS, IDX_ROWS_PER_STEP, 128)


_IDX3 = _gather_index()


def _sc_gather(x_rows, idx3):
    mesh = plsc.VectorSubcoreMesh(core_axis_name="c", subcore_axis_name="s")

    @functools.partial(
        pl.kernel,
        mesh=mesh,
        compiler_params=pltpu.CompilerParams(use_tc_tiling_on_sc=False),
        out_type=jax.ShapeDtypeStruct((NROWS, PS), jnp.float32),
        scratch_types=[
            pltpu.VMEM((IDX_ROWS_PER_STEP, 128), jnp.int32),
            pltpu.VMEM((ROWS_PER_STEP, PS), jnp.float32),
            pltpu.SemaphoreType.DMA,
        ],
    )
    def k(x_hbm, idx_hbm, out_hbm, idx_v, rows_v, sem):
        wid = lax.axis_index("s") * 2 + lax.axis_index("c")
        base = wid * RPW

        def step(t, carry):
            pltpu.sync_copy(idx_hbm.at[wid, t], idx_v)
            cps = [
                pltpu.async_copy(x_hbm.at[idx_v.at[j]],
                                 rows_v.at[pl.ds(j * 128, 128)], sem)
                for j in range(IDX_ROWS_PER_STEP)
            ]
            for cp in cps:
                cp.wait()
            pltpu.sync_copy(
                rows_v,
                out_hbm.at[pl.ds(base + t * ROWS_PER_STEP, ROWS_PER_STEP)])
            return carry

        lax.fori_loop(0, STEPS, step, 0)

    return k(x_rows, idx3)


def _tc_masked_copy(x3):
    def body(in_ref, out_ref):
        r = lax.broadcasted_iota(jnp.int32, (H, W), 0)
        c = lax.broadcasted_iota(jnp.int32, (H, W), 1)
        inside = (r >= R0) & (r < R1) & (c >= C0) & (c < C1)
        out_ref[0] = jnp.where(inside, 0.0, in_ref[0])

    return pl.pallas_call(
        body,
        grid=(B * C,),
        in_specs=[pl.BlockSpec((1, H, W), lambda g: (g, 0, 0))],
        out_specs=pl.BlockSpec((1, H, W), lambda g: (g, 0, 0)),
        out_shape=jax.ShapeDtypeStruct((B * C, H, W), jnp.float32),
    )(x3)


def kernel(x):
    x3 = x.reshape(B * C, H, W)
    masked_input = _tc_masked_copy(x3).reshape(B, C, H, W)

    x_rows = x.reshape(B * C * H * NPW, PS)
    rows = _sc_gather(x_rows, jnp.asarray(_IDX3))
    unmasked_patches = rows.reshape(B, NKEEP, C * PS * PS)

    mask = jnp.asarray(_MASK_CONST)
    return (masked_input, mask, jnp.asarray(unmasked_patches))
